# baseline (speedup 1.0000x reference)
import jax
import jax.numpy as jnp
from jax.experimental import pallas as pl
from jax.experimental.pallas import tpu as pltpu

N = 4096
D = 128
BM = 512


def _fused_gcn_kernel(adj_ref, diff_ref, bf_ref, bl_ref, w1_ref, b1_ref,
                      a1_ref, w2_ref, b2_ref, a2_ref, o_ref):
    agg1 = jnp.dot(adj_ref[...], bf_ref[...], preferred_element_type=jnp.float32)
    t1 = jnp.dot(agg1, w1_ref[...], preferred_element_type=jnp.float32) + b1_ref[...]
    agg2 = jnp.dot(diff_ref[...], bl_ref[...], preferred_element_type=jnp.float32)
    t2 = jnp.dot(agg2, w2_ref[...], preferred_element_type=jnp.float32) + b2_ref[...]
    a1 = a1_ref[0, 0]
    a2 = a2_ref[0, 0]
    o_ref[...] = (jnp.where(t1 >= 0, t1, a1 * t1)
                  + jnp.where(t2 >= 0, t2, a2 * t2))


def kernel(bf, bl, adj, diff, W1, b1, a1, W2, b2, a2):
    adj2 = adj.reshape(N, N)
    diff2 = diff.reshape(N, N)
    bf2 = bf.reshape(N, D)
    bl2 = bl.reshape(N, D)
    b1r = b1.reshape(1, D)
    b2r = b2.reshape(1, D)
    a1r = a1.reshape(1, 1)
    a2r = a2.reshape(1, 1)

    grid = (N // BM,)
    row_blk = pl.BlockSpec((BM, N), lambda i: (i, 0))
    const_nd = pl.BlockSpec((N, D), lambda i: (0, 0))
    const_dd = pl.BlockSpec((D, D), lambda i: (0, 0))
    const_1d = pl.BlockSpec((1, D), lambda i: (0, 0))
    const_11 = pl.BlockSpec((1, 1), lambda i: (0, 0))

    out = pl.pallas_call(
        _fused_gcn_kernel,
        grid=grid,
        in_specs=[row_blk, row_blk, const_nd, const_nd, const_dd, const_1d,
                  const_11, const_dd, const_1d, const_11],
        out_specs=pl.BlockSpec((BM, D), lambda i: (i, 0)),
        out_shape=jax.ShapeDtypeStruct((N, D), jnp.float32),
        compiler_params=pltpu.CompilerParams(
            dimension_semantics=("parallel",),
        ),
    )(adj2, diff2, bf2, bl2, W1, b1r, a1r, W2, b2r, a2r)
    return out.reshape(1, N, D)


# speedup vs baseline: 1.0024x; 1.0024x over previous
import jax
import jax.numpy as jnp
from jax.experimental import pallas as pl
from jax.experimental.pallas import tpu as pltpu

N = 4096
D = 128
BM = 512


def _fused_gcn_kernel(adj_ref, diff_ref, bf_ref, bl_ref, w1_ref, b1_ref,
                      a1_ref, w2_ref, b2_ref, a2_ref, o_ref):
    adj_bf = adj_ref[...].astype(jnp.bfloat16)
    diff_bf = diff_ref[...].astype(jnp.bfloat16)
    f1 = bf_ref[...].astype(jnp.bfloat16)
    f2 = bl_ref[...].astype(jnp.bfloat16)
    agg1 = jnp.dot(adj_bf, f1, preferred_element_type=jnp.float32)
    t1 = jnp.dot(agg1, w1_ref[...], preferred_element_type=jnp.float32) + b1_ref[...]
    agg2 = jnp.dot(diff_bf, f2, preferred_element_type=jnp.float32)
    t2 = jnp.dot(agg2, w2_ref[...], preferred_element_type=jnp.float32) + b2_ref[...]
    a1 = a1_ref[0, 0]
    a2 = a2_ref[0, 0]
    o_ref[...] = (jnp.where(t1 >= 0, t1, a1 * t1)
                  + jnp.where(t2 >= 0, t2, a2 * t2))


def kernel(bf, bl, adj, diff, W1, b1, a1, W2, b2, a2):
    adj2 = adj.reshape(N, N)
    diff2 = diff.reshape(N, N)
    bf2 = bf.reshape(N, D)
    bl2 = bl.reshape(N, D)
    b1r = b1.reshape(1, D)
    b2r = b2.reshape(1, D)
    a1r = a1.reshape(1, 1)
    a2r = a2.reshape(1, 1)

    grid = (N // BM,)
    row_blk = pl.BlockSpec((BM, N), lambda i: (i, 0))
    const_nd = pl.BlockSpec((N, D), lambda i: (0, 0))
    const_dd = pl.BlockSpec((D, D), lambda i: (0, 0))
    const_1d = pl.BlockSpec((1, D), lambda i: (0, 0))
    const_11 = pl.BlockSpec((1, 1), lambda i: (0, 0))

    out = pl.pallas_call(
        _fused_gcn_kernel,
        grid=grid,
        in_specs=[row_blk, row_blk, const_nd, const_nd, const_dd, const_1d,
                  const_11, const_dd, const_1d, const_11],
        out_specs=pl.BlockSpec((BM, D), lambda i: (i, 0)),
        out_shape=jax.ShapeDtypeStruct((N, D), jnp.float32),
        compiler_params=pltpu.CompilerParams(
            dimension_semantics=("parallel",),
        ),
    )(adj2, diff2, bf2, bl2, W1, b1r, a1r, W2, b2r, a2r)
    return out.reshape(1, N, D)


# speedup vs baseline: 1.0445x; 1.0420x over previous
import jax
import jax.numpy as jnp
from jax.experimental import pallas as pl
from jax.experimental.pallas import tpu as pltpu

N = 4096
D = 128
CHUNK = 256
NBUF = 4
NCHUNKS = N // CHUNK


def _fused_gcn_kernel(bf_ref, bl_ref, w1_ref, b1_ref, a1_ref, w2_ref, b2_ref,
                      a2_ref, adj_hbm, diff_hbm, o_ref, abuf, dbuf, sems):
    f1 = bf_ref[...].astype(jnp.bfloat16)
    f2 = bl_ref[...].astype(jnp.bfloat16)
    a1 = a1_ref[0, 0]
    a2 = a2_ref[0, 0]

    def copy_pair(i):
        slot = i % NBUF
        a_cp = pltpu.make_async_copy(
            adj_hbm.at[pl.ds(i * CHUNK, CHUNK), :], abuf.at[slot],
            sems.at[0, slot])
        d_cp = pltpu.make_async_copy(
            diff_hbm.at[pl.ds(i * CHUNK, CHUNK), :], dbuf.at[slot],
            sems.at[1, slot])
        return a_cp, d_cp

    for i in range(NBUF):
        a_cp, d_cp = copy_pair(i)
        a_cp.start()
        d_cp.start()

    for i in range(NCHUNKS):
        slot = i % NBUF
        a_cp, d_cp = copy_pair(i)
        a_cp.wait()
        agg1 = jnp.dot(abuf[slot].astype(jnp.bfloat16), f1,
                       preferred_element_type=jnp.float32)
        d_cp.wait()
        agg2 = jnp.dot(dbuf[slot].astype(jnp.bfloat16), f2,
                       preferred_element_type=jnp.float32)
        if i + NBUF < NCHUNKS:
            a_nxt, d_nxt = copy_pair(i + NBUF)
            a_nxt.start()
            d_nxt.start()
        t1 = jnp.dot(agg1, w1_ref[...], preferred_element_type=jnp.float32) + b1_ref[...]
        t2 = jnp.dot(agg2, w2_ref[...], preferred_element_type=jnp.float32) + b2_ref[...]
        o_ref[pl.ds(i * CHUNK, CHUNK), :] = (
            jnp.where(t1 >= 0, t1, a1 * t1) + jnp.where(t2 >= 0, t2, a2 * t2))


def kernel(bf, bl, adj, diff, W1, b1, a1, W2, b2, a2):
    adj2 = adj.reshape(N, N)
    diff2 = diff.reshape(N, N)
    bf2 = bf.reshape(N, D)
    bl2 = bl.reshape(N, D)
    b1r = b1.reshape(1, D)
    b2r = b2.reshape(1, D)
    a1r = a1.reshape(1, 1)
    a2r = a2.reshape(1, 1)

    vmem = pl.BlockSpec(memory_space=pltpu.MemorySpace.VMEM)
    hbm = pl.BlockSpec(memory_space=pltpu.MemorySpace.HBM)

    out = pl.pallas_call(
        _fused_gcn_kernel,
        in_specs=[vmem, vmem, vmem, vmem, vmem, vmem, vmem, vmem, hbm, hbm],
        out_specs=vmem,
        out_shape=jax.ShapeDtypeStruct((N, D), jnp.float32),
        scratch_shapes=[
            pltpu.VMEM((NBUF, CHUNK, N), jnp.float32),
            pltpu.VMEM((NBUF, CHUNK, N), jnp.float32),
            pltpu.SemaphoreType.DMA((2, NBUF)),
        ],
    )(bf2, bl2, W1, b1r, a1r, W2, b2r, a2r, adj2, diff2)
    return out.reshape(1, N, D)
